# baseline (speedup 1.0000x reference)
import functools

import jax
import jax.numpy as jnp
from jax import lax
from jax.experimental import pallas as pl
from jax.experimental.pallas import tpu as pltpu
from jax.experimental.pallas import tpu_sc as plsc

S, HS = 2048, 1024
E, TOPK = 8, 2
DFF, DSH = 1408, 2816

NPAIR = S * TOPK
BM = 128
NB = 40
NBP = 48
NPAD = NB * BM
NC, NS, L = 2, 16, 16
NW = NC * NS
_SC_MESH = plsc.VectorSubcoreMesh(core_axis_name="c", subcore_axis_name="s")
_SC_PARAMS = pltpu.CompilerParams(needs_layout_passes=False)


def _wid():
    return lax.axis_index("s") * NC + lax.axis_index("c")


def _router_body(x_ref, gw_ref, logits_ref, w_ref, eids_ref):
    x = x_ref[...]
    logits = jnp.dot(x, gw_ref[...], preferred_element_type=jnp.float32)
    logits_ref[...] = logits
    p = jax.nn.softmax(logits, axis=-1)
    iota = jax.lax.broadcasted_iota(jnp.int32, p.shape, 1)
    m1 = jnp.max(p, axis=-1, keepdims=True)
    e1 = jnp.min(jnp.where(p == m1, iota, E), axis=-1, keepdims=True)
    p2 = jnp.where(iota == e1, -jnp.inf, p)
    m2 = jnp.max(p2, axis=-1, keepdims=True)
    e2 = jnp.min(jnp.where(p2 == m2, iota, E), axis=-1, keepdims=True)
    tot = m1 + m2
    w_ref[...] = jnp.concatenate([m1 / tot, m2 / tot], axis=-1)
    eids_ref[...] = jnp.concatenate([e1, e2], axis=-1)


def _router(x, gate_w):
    return pl.pallas_call(
        _router_body,
        out_shape=(
            jax.ShapeDtypeStruct((S, E), jnp.float32),
            jax.ShapeDtypeStruct((S, TOPK), jnp.float32),
            jax.ShapeDtypeStruct((S, TOPK), jnp.int32),
        ),
    )(x, gate_w)


def _meta_body(ef_hbm, wf_hbm, gidx_hbm, ws_hbm, pos_hbm, bexp_hbm,
               ev, wv, gidxv, wsv, posv, bexpv):
    @pl.when(_wid() == 0)
    def _():
        pltpu.sync_copy(ef_hbm, ev)
        pltpu.sync_copy(wf_hbm, wv)
        zi = jnp.zeros((L,), jnp.int32)
        zf = jnp.zeros((L,), jnp.float32)
        lane = lax.iota(jnp.int32, L)

        def init(i, c):
            gidxv[pl.ds(i * L, L)] = zi
            wsv[pl.ds(i * L, L)] = zf
            return c
        lax.fori_loop(0, NPAD // L, init, 0)

        def cnt_step(j, cs):
            v = ev[pl.ds(j * L, L)]
            return tuple(cs[e] + jnp.where(v == e, 1, 0) for e in range(E))
        partial = lax.fori_loop(0, NPAIR // L, cnt_step, (zi,) * E)
        counts = [jnp.sum(partial[e]) for e in range(E)]

        nb = [(counts[e] + (BM - 1)) >> 7 for e in range(E)]
        off, offb = [], []
        acc = jnp.int32(0)
        accb = jnp.int32(0)
        for e in range(E):
            off.append(acc)
            offb.append(accb)
            acc = acc + (nb[e] << 7)
            accb = accb + nb[e]
        total_b = accb

        for r in range(NBP // L):
            bid = lane + r * L
            be = zi
            for e in range(E):
                be = be + jnp.where(bid >= offb[e] + nb[e], 1, 0)
            bexpv[pl.ds(r * L, L)] = jnp.where(bid >= total_b, -1, be)

        def p2_step(j, bases):
            v = ev[pl.ds(j * L, L)]
            wj = wv[pl.ds(j * L, L)]
            pos_vec = zi
            newb = list(bases)
            for e in range(E):
                m = v == e
                mi = jnp.where(m, 1, 0)
                pref = plsc.cumsum(mi)
                pos_vec = jnp.where(m, newb[e] + pref - 1, pos_vec)
                newb[e] = newb[e] + jnp.sum(mi)
            posv[pl.ds(j * L, L)] = pos_vec
            token = (lane + j * L) & (S - 1)
            plsc.store_scatter(gidxv, [pos_vec], token)
            plsc.store_scatter(wsv, [pos_vec], wj)
            return tuple(newb)
        lax.fori_loop(0, NPAIR // L, p2_step, tuple(off))

        pltpu.sync_copy(gidxv, gidx_hbm)
        pltpu.sync_copy(wsv, ws_hbm)
        pltpu.sync_copy(posv, pos_hbm)
        pltpu.sync_copy(bexpv, bexp_hbm)


def _meta(eflat, wflat):
    return pl.kernel(
        _meta_body,
        out_type=(
            jax.ShapeDtypeStruct((NPAD,), jnp.int32),
            jax.ShapeDtypeStruct((NPAD,), jnp.float32),
            jax.ShapeDtypeStruct((NPAIR,), jnp.int32),
            jax.ShapeDtypeStruct((NBP,), jnp.int32),
        ),
        mesh=_SC_MESH,
        compiler_params=_SC_PARAMS,
        scratch_types=[
            pltpu.VMEM((NPAIR,), jnp.int32),
            pltpu.VMEM((NPAIR,), jnp.float32),
            pltpu.VMEM((NPAD,), jnp.int32),
            pltpu.VMEM((NPAD,), jnp.float32),
            pltpu.VMEM((NPAIR,), jnp.int32),
            pltpu.VMEM((NBP,), jnp.int32),
        ],
    )(eflat, wflat)


_GROWS = NPAD // NW
_GCH = 40


def _gather_body(x_hbm, gidx_hbm, xs_hbm, idxv, rowsv, sem):
    base = _wid() * _GROWS

    def step(c, carry):
        pltpu.sync_copy(gidx_hbm.at[pl.ds(base + c * _GCH, _GCH)], idxv)
        pltpu.async_copy(x_hbm.at[idxv], rowsv, sem).wait()
        pltpu.sync_copy(rowsv, xs_hbm.at[pl.ds(base + c * _GCH, _GCH)])
        return carry
    lax.fori_loop(0, _GROWS // _GCH, step, 0)


def _gather(x, gidx):
    return pl.kernel(
        _gather_body,
        out_type=jax.ShapeDtypeStruct((NPAD, HS), jnp.float32),
        mesh=_SC_MESH,
        compiler_params=_SC_PARAMS,
        scratch_types=[
            pltpu.VMEM((_GCH,), jnp.int32),
            pltpu.VMEM((_GCH, HS), jnp.float32),
            pltpu.SemaphoreType.DMA,
        ],
    )(x, gidx)


def _group_body(bexp_ref, xs_ref, wg_ref, wu_ref, wd_ref, ws_ref, ys_ref):
    e = bexp_ref[pl.program_id(0)]

    @pl.when(e >= 0)
    def _():
        x = xs_ref[...].astype(jnp.bfloat16)
        g = jnp.dot(x, wg_ref[0].astype(jnp.bfloat16),
                    preferred_element_type=jnp.float32)
        u = jnp.dot(x, wu_ref[0].astype(jnp.bfloat16),
                    preferred_element_type=jnp.float32)
        h = ((g * jax.nn.sigmoid(g)) * u).astype(jnp.bfloat16)
        o = jnp.dot(h, wd_ref[0].astype(jnp.bfloat16),
                    preferred_element_type=jnp.float32)
        ys_ref[...] = ws_ref[...] * o


def _grouped(bexp, xs, gate_proj_w, up_proj_w, down_proj_w, ws):
    grid_spec = pltpu.PrefetchScalarGridSpec(
        num_scalar_prefetch=1,
        grid=(NB,),
        in_specs=[
            pl.BlockSpec((BM, HS), lambda b, bexp: (b, 0)),
            pl.BlockSpec((1, HS, DFF),
                         lambda b, bexp: (jnp.maximum(bexp[b], 0), 0, 0)),
            pl.BlockSpec((1, HS, DFF),
                         lambda b, bexp: (jnp.maximum(bexp[b], 0), 0, 0)),
            pl.BlockSpec((1, DFF, HS),
                         lambda b, bexp: (jnp.maximum(bexp[b], 0), 0, 0)),
            pl.BlockSpec((BM, 1), lambda b, bexp: (b, 0)),
        ],
        out_specs=pl.BlockSpec((BM, HS), lambda b, bexp: (b, 0)),
    )
    return pl.pallas_call(
        _group_body,
        grid_spec=grid_spec,
        out_shape=jax.ShapeDtypeStruct((NPAD, HS), jnp.float32),
    )(bexp, xs, gate_proj_w, up_proj_w, down_proj_w, ws)


def _shared_body(x_ref, wg_ref, wu_ref, wd_ref, sg_ref, out_ref):
    d = pl.program_id(0)
    t = pl.program_id(1)
    nd = pl.num_programs(0)
    bt = x_ref.shape[0]

    x = x_ref[...].astype(jnp.bfloat16)
    g = jnp.dot(x, wg_ref[...].astype(jnp.bfloat16),
                preferred_element_type=jnp.float32)
    u = jnp.dot(x, wu_ref[...].astype(jnp.bfloat16),
                preferred_element_type=jnp.float32)
    h = ((g * jax.nn.sigmoid(g)) * u).astype(jnp.bfloat16)
    y = jnp.dot(h, wd_ref[...].astype(jnp.bfloat16),
                preferred_element_type=jnp.float32)
    sl = pl.ds(t * bt, bt)

    @pl.when(d == 0)
    def _():
        out_ref[sl, :] = y

    @pl.when(d > 0)
    def _():
        out_ref[sl, :] += y

    @pl.when(d == nd - 1)
    def _():
        gate = jax.nn.sigmoid(
            jnp.dot(x_ref[...], sg_ref[...],
                    preferred_element_type=jnp.float32))
        out_ref[sl, :] *= gate


def _shared(x, wg, wu, wd, sgw, bt=512, bd=1408):
    nt, nd = S // bt, DSH // bd
    return pl.pallas_call(
        _shared_body,
        grid=(nd, nt),
        in_specs=[
            pl.BlockSpec((bt, HS), lambda d, t: (t, 0)),
            pl.BlockSpec((HS, bd), lambda d, t: (0, d)),
            pl.BlockSpec((HS, bd), lambda d, t: (0, d)),
            pl.BlockSpec((bd, HS), lambda d, t: (d, 0)),
            pl.BlockSpec((HS, 1), lambda d, t: (0, 0)),
        ],
        out_specs=pl.BlockSpec((S, HS), lambda d, t: (0, 0)),
        out_shape=jax.ShapeDtypeStruct((S, HS), jnp.float32),
    )(x, wg, wu, wd, sgw)


_TPW = S // NW
_CCH = 16


def _combine_body(ys_hbm, sh_hbm, pos_hbm, out_hbm, p0v, p1v, y0v, y1v, shv,
                  sem):
    tb = _wid() * _TPW

    def chunk(c, carry):
        b = tb + c * _CCH
        pltpu.sync_copy(pos_hbm.at[pl.ds(b, _CCH)], p0v)
        pltpu.sync_copy(pos_hbm.at[pl.ds(S + b, _CCH)], p1v)
        cp0 = pltpu.async_copy(ys_hbm.at[p0v], y0v, sem)
        cp1 = pltpu.async_copy(ys_hbm.at[p1v], y1v, sem)
        pltpu.sync_copy(sh_hbm.at[pl.ds(b, _CCH)], shv)
        cp0.wait()
        cp1.wait()

        def dstep(d, carry2):
            sl = pl.ds(d * L, L)
            for i in range(_CCH):
                shv[i, sl] += y0v[i, sl] + y1v[i, sl]
            return carry2
        lax.fori_loop(0, HS // L, dstep, 0)
        pltpu.sync_copy(shv, out_hbm.at[pl.ds(b, _CCH)])
        return carry
    lax.fori_loop(0, _TPW // _CCH, chunk, 0)


def _combine(ys, sh, pos):
    return pl.kernel(
        _combine_body,
        out_type=jax.ShapeDtypeStruct((S, HS), jnp.float32),
        mesh=_SC_MESH,
        compiler_params=_SC_PARAMS,
        scratch_types=[
            pltpu.VMEM((_CCH,), jnp.int32),
            pltpu.VMEM((_CCH,), jnp.int32),
            pltpu.VMEM((_CCH, HS), jnp.float32),
            pltpu.VMEM((_CCH, HS), jnp.float32),
            pltpu.VMEM((_CCH, HS), jnp.float32),
            pltpu.SemaphoreType.DMA,
        ],
    )(ys, sh, pos)


def kernel(hidden_states, gate_w, gate_proj_w, up_proj_w, down_proj_w,
           shared_gate_proj_w, shared_up_proj_w, shared_down_proj_w,
           shared_expert_gate_w):
    x = hidden_states.reshape(S, HS)
    logits, w2, e2 = _router(x, gate_w)
    eflat = e2.T.reshape(NPAIR)
    wflat = w2.T.reshape(NPAIR)
    gidx, ws, pos, bexp = _meta(eflat, wflat)
    xs = _gather(x, gidx)
    ys = _grouped(bexp, xs, gate_proj_w, up_proj_w, down_proj_w,
                  ws.reshape(NPAD, 1))
    sh = _shared(x, shared_gate_proj_w, shared_up_proj_w, shared_down_proj_w,
                 shared_expert_gate_w)
    out = _combine(ys, sh, pos)
    return (out.reshape(1, S, HS), logits.reshape(1, S, E))


# speedup vs baseline: 1.0368x; 1.0368x over previous
import functools

import jax
import jax.numpy as jnp
from jax import lax
from jax.experimental import pallas as pl
from jax.experimental.pallas import tpu as pltpu
from jax.experimental.pallas import tpu_sc as plsc

S, HS = 2048, 1024
E, TOPK = 8, 2
DFF, DSH = 1408, 2816

NPAIR = S * TOPK
BM = 128
NB = 40
NBP = 48
NPAD = NB * BM
NC, NS, L = 2, 16, 16
NW = NC * NS
_SC_MESH = plsc.VectorSubcoreMesh(core_axis_name="c", subcore_axis_name="s")
_SC_PARAMS = pltpu.CompilerParams(needs_layout_passes=False)


def _wid():
    return lax.axis_index("s") * NC + lax.axis_index("c")


def _router_body(x_ref, gw_ref, logits_ref, w_ref, eids_ref):
    x = x_ref[...]
    logits = jnp.dot(x, gw_ref[...], preferred_element_type=jnp.float32)
    logits_ref[...] = logits
    p = jax.nn.softmax(logits, axis=-1)
    iota = jax.lax.broadcasted_iota(jnp.int32, p.shape, 1)
    m1 = jnp.max(p, axis=-1, keepdims=True)
    e1 = jnp.min(jnp.where(p == m1, iota, E), axis=-1, keepdims=True)
    p2 = jnp.where(iota == e1, -jnp.inf, p)
    m2 = jnp.max(p2, axis=-1, keepdims=True)
    e2 = jnp.min(jnp.where(p2 == m2, iota, E), axis=-1, keepdims=True)
    tot = m1 + m2
    w_ref[...] = jnp.concatenate([m1 / tot, m2 / tot], axis=-1)
    eids_ref[...] = jnp.concatenate([e1, e2], axis=-1)


def _router(x, gate_w):
    return pl.pallas_call(
        _router_body,
        out_shape=(
            jax.ShapeDtypeStruct((S, E), jnp.float32),
            jax.ShapeDtypeStruct((S, TOPK), jnp.float32),
            jax.ShapeDtypeStruct((S, TOPK), jnp.int32),
        ),
    )(x, gate_w)


def _meta_body(ef_hbm, wf_hbm, gidx_hbm, ws_hbm, pos_hbm, bexp_hbm,
               ev, wv, gidxv, wsv, posv, bexpv):
    @pl.when(_wid() == 0)
    def _():
        pltpu.sync_copy(ef_hbm, ev)
        pltpu.sync_copy(wf_hbm, wv)
        zi = jnp.zeros((L,), jnp.int32)
        zf = jnp.zeros((L,), jnp.float32)
        lane = lax.iota(jnp.int32, L)

        def init(i, c):
            gidxv[pl.ds(i * L, L)] = zi
            wsv[pl.ds(i * L, L)] = zf
            return c
        lax.fori_loop(0, NPAD // L, init, 0)

        def cnt_step(j, cs):
            v = ev[pl.ds(j * L, L)]
            return tuple(cs[e] + jnp.where(v == e, 1, 0) for e in range(E))
        partial = lax.fori_loop(0, NPAIR // L, cnt_step, (zi,) * E)
        counts = [jnp.sum(partial[e]) for e in range(E)]

        nb = [(counts[e] + (BM - 1)) >> 7 for e in range(E)]
        off, offb = [], []
        acc = jnp.int32(0)
        accb = jnp.int32(0)
        for e in range(E):
            off.append(acc)
            offb.append(accb)
            acc = acc + (nb[e] << 7)
            accb = accb + nb[e]
        total_b = accb

        for r in range(NBP // L):
            bid = lane + r * L
            be = zi
            for e in range(E):
                be = be + jnp.where(bid >= offb[e] + nb[e], 1, 0)
            bexpv[pl.ds(r * L, L)] = jnp.where(bid >= total_b, -1, be)

        def p2_step(j, bases):
            v = ev[pl.ds(j * L, L)]
            wj = wv[pl.ds(j * L, L)]
            pos_vec = zi
            newb = list(bases)
            for e in range(E):
                m = v == e
                mi = jnp.where(m, 1, 0)
                pref = plsc.cumsum(mi)
                pos_vec = jnp.where(m, newb[e] + pref - 1, pos_vec)
                newb[e] = newb[e] + jnp.sum(mi)
            posv[pl.ds(j * L, L)] = pos_vec
            token = (lane + j * L) & (S - 1)
            plsc.store_scatter(gidxv, [pos_vec], token)
            plsc.store_scatter(wsv, [pos_vec], wj)
            return tuple(newb)
        lax.fori_loop(0, NPAIR // L, p2_step, tuple(off))

        pltpu.sync_copy(gidxv, gidx_hbm)
        pltpu.sync_copy(wsv, ws_hbm)
        pltpu.sync_copy(posv, pos_hbm)
        pltpu.sync_copy(bexpv, bexp_hbm)


def _meta(eflat, wflat):
    return pl.kernel(
        _meta_body,
        out_type=(
            jax.ShapeDtypeStruct((NPAD,), jnp.int32),
            jax.ShapeDtypeStruct((NPAD,), jnp.float32),
            jax.ShapeDtypeStruct((NPAIR,), jnp.int32),
            jax.ShapeDtypeStruct((NBP,), jnp.int32),
        ),
        mesh=_SC_MESH,
        compiler_params=_SC_PARAMS,
        scratch_types=[
            pltpu.VMEM((NPAIR,), jnp.int32),
            pltpu.VMEM((NPAIR,), jnp.float32),
            pltpu.VMEM((NPAD,), jnp.int32),
            pltpu.VMEM((NPAD,), jnp.float32),
            pltpu.VMEM((NPAIR,), jnp.int32),
            pltpu.VMEM((NBP,), jnp.int32),
        ],
    )(eflat, wflat)


_GROWS = NPAD // NW
_GCH = 40


def _gather_body(x_hbm, gidx_hbm, xs_hbm, idxv, r0, r1, gsem, wsem):
    base = _wid() * _GROWS
    pltpu.sync_copy(gidx_hbm.at[pl.ds(base, _GROWS)], idxv)
    bufs = (r0, r1)
    nch = _GROWS // _GCH
    g = [None] * nch
    w = [None] * nch
    g[0] = pltpu.async_copy(x_hbm.at[idxv.at[pl.ds(0, _GCH)]], bufs[0], gsem)
    for c in range(nch):
        if c + 1 < nch:
            if c >= 1:
                w[c - 1].wait()
            g[c + 1] = pltpu.async_copy(
                x_hbm.at[idxv.at[pl.ds((c + 1) * _GCH, _GCH)]],
                bufs[(c + 1) % 2], gsem)
        g[c].wait()
        w[c] = pltpu.async_copy(bufs[c % 2],
                                xs_hbm.at[pl.ds(base + c * _GCH, _GCH)], wsem)
    w[nch - 2].wait()
    w[nch - 1].wait()


def _gather(x, gidx):
    return pl.kernel(
        _gather_body,
        out_type=jax.ShapeDtypeStruct((NPAD, HS), jnp.float32),
        mesh=_SC_MESH,
        compiler_params=_SC_PARAMS,
        scratch_types=[
            pltpu.VMEM((_GROWS,), jnp.int32),
            pltpu.VMEM((_GCH, HS), jnp.float32),
            pltpu.VMEM((_GCH, HS), jnp.float32),
            pltpu.SemaphoreType.DMA,
            pltpu.SemaphoreType.DMA,
        ],
    )(x, gidx)


def _group_body(bexp_ref, xs_ref, wg_ref, wu_ref, wd_ref, ws_ref, ys_ref):
    e = bexp_ref[pl.program_id(0)]

    @pl.when(e >= 0)
    def _():
        x = xs_ref[...].astype(jnp.bfloat16)
        g = jnp.dot(x, wg_ref[0].astype(jnp.bfloat16),
                    preferred_element_type=jnp.float32)
        u = jnp.dot(x, wu_ref[0].astype(jnp.bfloat16),
                    preferred_element_type=jnp.float32)
        h = ((g * jax.nn.sigmoid(g)) * u).astype(jnp.bfloat16)
        o = jnp.dot(h, wd_ref[0].astype(jnp.bfloat16),
                    preferred_element_type=jnp.float32)
        ys_ref[...] = ws_ref[...] * o


def _grouped(bexp, xs, gate_proj_w, up_proj_w, down_proj_w, ws):
    grid_spec = pltpu.PrefetchScalarGridSpec(
        num_scalar_prefetch=1,
        grid=(NB,),
        in_specs=[
            pl.BlockSpec((BM, HS), lambda b, bexp: (b, 0)),
            pl.BlockSpec((1, HS, DFF),
                         lambda b, bexp: (jnp.maximum(bexp[b], 0), 0, 0)),
            pl.BlockSpec((1, HS, DFF),
                         lambda b, bexp: (jnp.maximum(bexp[b], 0), 0, 0)),
            pl.BlockSpec((1, DFF, HS),
                         lambda b, bexp: (jnp.maximum(bexp[b], 0), 0, 0)),
            pl.BlockSpec((BM, 1), lambda b, bexp: (b, 0)),
        ],
        out_specs=pl.BlockSpec((BM, HS), lambda b, bexp: (b, 0)),
    )
    return pl.pallas_call(
        _group_body,
        grid_spec=grid_spec,
        out_shape=jax.ShapeDtypeStruct((NPAD, HS), jnp.float32),
    )(bexp, xs, gate_proj_w, up_proj_w, down_proj_w, ws)


def _shared_body(x_ref, wg_ref, wu_ref, wd_ref, sg_ref, out_ref):
    d = pl.program_id(0)
    t = pl.program_id(1)
    nd = pl.num_programs(0)
    bt = x_ref.shape[0]

    x = x_ref[...].astype(jnp.bfloat16)
    g = jnp.dot(x, wg_ref[...].astype(jnp.bfloat16),
                preferred_element_type=jnp.float32)
    u = jnp.dot(x, wu_ref[...].astype(jnp.bfloat16),
                preferred_element_type=jnp.float32)
    h = ((g * jax.nn.sigmoid(g)) * u).astype(jnp.bfloat16)
    y = jnp.dot(h, wd_ref[...].astype(jnp.bfloat16),
                preferred_element_type=jnp.float32)
    sl = pl.ds(t * bt, bt)

    @pl.when(d == 0)
    def _():
        out_ref[sl, :] = y

    @pl.when(d > 0)
    def _():
        out_ref[sl, :] += y

    @pl.when(d == nd - 1)
    def _():
        gate = jax.nn.sigmoid(
            jnp.dot(x_ref[...], sg_ref[...],
                    preferred_element_type=jnp.float32))
        out_ref[sl, :] *= gate


def _shared(x, wg, wu, wd, sgw, bt=512, bd=1408):
    nt, nd = S // bt, DSH // bd
    return pl.pallas_call(
        _shared_body,
        grid=(nd, nt),
        in_specs=[
            pl.BlockSpec((bt, HS), lambda d, t: (t, 0)),
            pl.BlockSpec((HS, bd), lambda d, t: (0, d)),
            pl.BlockSpec((HS, bd), lambda d, t: (0, d)),
            pl.BlockSpec((bd, HS), lambda d, t: (d, 0)),
            pl.BlockSpec((HS, 1), lambda d, t: (0, 0)),
        ],
        out_specs=pl.BlockSpec((S, HS), lambda d, t: (0, 0)),
        out_shape=jax.ShapeDtypeStruct((S, HS), jnp.float32),
    )(x, wg, wu, wd, sgw)


_TPW = S // NW
_CCH = 16


def _combine_body(ys_hbm, sh_hbm, pos_hbm, out_hbm, p0v, p1v,
                  y00, y01, y10, y11, sh0, sh1, gsem, wsem):
    tb = _wid() * _TPW
    pltpu.sync_copy(pos_hbm.at[pl.ds(tb, _TPW)], p0v)
    pltpu.sync_copy(pos_hbm.at[pl.ds(S + tb, _TPW)], p1v)
    y0b = (y00, y01)
    y1b = (y10, y11)
    shb = (sh0, sh1)
    nch = _TPW // _CCH
    g = [None] * nch
    w = [None] * nch

    def issue(c):
        sl = pl.ds(c * _CCH, _CCH)
        a = pltpu.async_copy(ys_hbm.at[p0v.at[sl]], y0b[c % 2], gsem)
        b = pltpu.async_copy(ys_hbm.at[p1v.at[sl]], y1b[c % 2], gsem)
        d = pltpu.async_copy(sh_hbm.at[pl.ds(tb + c * _CCH, _CCH)],
                             shb[c % 2], gsem)
        return (a, b, d)

    g[0] = issue(0)
    for c in range(nch):
        if c + 1 < nch:
            if c >= 1:
                w[c - 1].wait()
            g[c + 1] = issue(c + 1)
        for cp in g[c]:
            cp.wait()

        def dstep(d, carry2):
            sl = pl.ds(d * L, L)
            for i in range(_CCH):
                shb[c % 2][i, sl] += y0b[c % 2][i, sl] + y1b[c % 2][i, sl]
            return carry2
        lax.fori_loop(0, HS // L, dstep, 0)
        w[c] = pltpu.async_copy(shb[c % 2],
                                out_hbm.at[pl.ds(tb + c * _CCH, _CCH)], wsem)
    w[nch - 2].wait()
    w[nch - 1].wait()


def _combine(ys, sh, pos):
    return pl.kernel(
        _combine_body,
        out_type=jax.ShapeDtypeStruct((S, HS), jnp.float32),
        mesh=_SC_MESH,
        compiler_params=_SC_PARAMS,
        scratch_types=[
            pltpu.VMEM((_TPW,), jnp.int32),
            pltpu.VMEM((_TPW,), jnp.int32),
            pltpu.VMEM((_CCH, HS), jnp.float32),
            pltpu.VMEM((_CCH, HS), jnp.float32),
            pltpu.VMEM((_CCH, HS), jnp.float32),
            pltpu.VMEM((_CCH, HS), jnp.float32),
            pltpu.VMEM((_CCH, HS), jnp.float32),
            pltpu.VMEM((_CCH, HS), jnp.float32),
            pltpu.SemaphoreType.DMA,
            pltpu.SemaphoreType.DMA,
        ],
    )(ys, sh, pos)


def kernel(hidden_states, gate_w, gate_proj_w, up_proj_w, down_proj_w,
           shared_gate_proj_w, shared_up_proj_w, shared_down_proj_w,
           shared_expert_gate_w):
    x = hidden_states.reshape(S, HS)
    logits, w2, e2 = _router(x, gate_w)
    eflat = e2.T.reshape(NPAIR)
    wflat = w2.T.reshape(NPAIR)
    gidx, ws, pos, bexp = _meta(eflat, wflat)
    xs = _gather(x, gidx)
    ys = _grouped(bexp, xs, gate_proj_w, up_proj_w, down_proj_w,
                  ws.reshape(NPAD, 1))
    sh = _shared(x, shared_gate_proj_w, shared_up_proj_w, shared_down_proj_w,
                 shared_expert_gate_w)
    out = _combine(ys, sh, pos)
    return (out.reshape(1, S, HS), logits.reshape(1, S, E))


# speedup vs baseline: 1.0408x; 1.0039x over previous
import functools

import jax
import jax.numpy as jnp
from jax import lax
from jax.experimental import pallas as pl
from jax.experimental.pallas import tpu as pltpu
from jax.experimental.pallas import tpu_sc as plsc

S, HS = 2048, 1024
E, TOPK = 8, 2
DFF, DSH = 1408, 2816

NPAIR = S * TOPK
BM = 128
NB = 40
NBP = 48
NPAD = NB * BM
NC, NS, L = 2, 16, 16
NW = NC * NS
_SC_MESH = plsc.VectorSubcoreMesh(core_axis_name="c", subcore_axis_name="s")
_SC_PARAMS = pltpu.CompilerParams(needs_layout_passes=False)


def _wid():
    return lax.axis_index("s") * NC + lax.axis_index("c")


def _router_body(x_ref, gw_ref, logits_ref, w_ref, eids_ref):
    x = x_ref[...]
    logits = jnp.dot(x, gw_ref[...], preferred_element_type=jnp.float32)
    logits_ref[...] = logits
    p = jax.nn.softmax(logits, axis=-1)
    iota = jax.lax.broadcasted_iota(jnp.int32, p.shape, 1)
    m1 = jnp.max(p, axis=-1, keepdims=True)
    e1 = jnp.min(jnp.where(p == m1, iota, E), axis=-1, keepdims=True)
    p2 = jnp.where(iota == e1, -jnp.inf, p)
    m2 = jnp.max(p2, axis=-1, keepdims=True)
    e2 = jnp.min(jnp.where(p2 == m2, iota, E), axis=-1, keepdims=True)
    tot = m1 + m2
    w_ref[...] = jnp.concatenate([m1 / tot, m2 / tot], axis=-1)
    eids_ref[...] = jnp.concatenate([e1, e2], axis=-1)


def _router(x, gate_w):
    return pl.pallas_call(
        _router_body,
        out_shape=(
            jax.ShapeDtypeStruct((S, E), jnp.float32),
            jax.ShapeDtypeStruct((S, TOPK), jnp.float32),
            jax.ShapeDtypeStruct((S, TOPK), jnp.int32),
        ),
    )(x, gate_w)


def _meta_body(ef_hbm, wf_hbm, gidx_hbm, ws_hbm, pos_hbm, bexp_hbm,
               ev, wv, gidxv, wsv, posv, bexpv):
    @pl.when(_wid() == 0)
    def _():
        pltpu.sync_copy(ef_hbm, ev)
        pltpu.sync_copy(wf_hbm, wv)
        zi = jnp.zeros((L,), jnp.int32)
        zf = jnp.zeros((L,), jnp.float32)
        lane = lax.iota(jnp.int32, L)

        def init(i, c):
            gidxv[pl.ds(i * L, L)] = zi
            wsv[pl.ds(i * L, L)] = zf
            return c
        lax.fori_loop(0, NPAD // L, init, 0)

        def cnt_step(j, cs):
            v = ev[pl.ds(j * L, L)]
            return tuple(cs[e] + jnp.where(v == e, 1, 0) for e in range(E))
        partial = lax.fori_loop(0, NPAIR // L, cnt_step, (zi,) * E)
        counts = [jnp.sum(partial[e]) for e in range(E)]

        nb = [(counts[e] + (BM - 1)) >> 7 for e in range(E)]
        off, offb = [], []
        acc = jnp.int32(0)
        accb = jnp.int32(0)
        for e in range(E):
            off.append(acc)
            offb.append(accb)
            acc = acc + (nb[e] << 7)
            accb = accb + nb[e]
        total_b = accb

        for r in range(NBP // L):
            bid = lane + r * L
            be = zi
            for e in range(E):
                be = be + jnp.where(bid >= offb[e] + nb[e], 1, 0)
            bexpv[pl.ds(r * L, L)] = jnp.where(bid >= total_b, -1, be)

        def p2_step(j, bases):
            v = ev[pl.ds(j * L, L)]
            wj = wv[pl.ds(j * L, L)]
            pos_vec = zi
            newb = list(bases)
            for e in range(E):
                m = v == e
                mi = jnp.where(m, 1, 0)
                pref = plsc.cumsum(mi)
                pos_vec = jnp.where(m, newb[e] + pref - 1, pos_vec)
                newb[e] = newb[e] + jnp.sum(mi)
            posv[pl.ds(j * L, L)] = pos_vec
            token = (lane + j * L) & (S - 1)
            plsc.store_scatter(gidxv, [pos_vec], token)
            plsc.store_scatter(wsv, [pos_vec], wj)
            return tuple(newb)
        lax.fori_loop(0, NPAIR // L, p2_step, tuple(off))

        pltpu.sync_copy(gidxv, gidx_hbm)
        pltpu.sync_copy(wsv, ws_hbm)
        pltpu.sync_copy(posv, pos_hbm)
        pltpu.sync_copy(bexpv, bexp_hbm)


def _meta(eflat, wflat):
    return pl.kernel(
        _meta_body,
        out_type=(
            jax.ShapeDtypeStruct((NPAD,), jnp.int32),
            jax.ShapeDtypeStruct((NPAD,), jnp.float32),
            jax.ShapeDtypeStruct((NPAIR,), jnp.int32),
            jax.ShapeDtypeStruct((NBP,), jnp.int32),
        ),
        mesh=_SC_MESH,
        compiler_params=_SC_PARAMS,
        scratch_types=[
            pltpu.VMEM((NPAIR,), jnp.int32),
            pltpu.VMEM((NPAIR,), jnp.float32),
            pltpu.VMEM((NPAD,), jnp.int32),
            pltpu.VMEM((NPAD,), jnp.float32),
            pltpu.VMEM((NPAIR,), jnp.int32),
            pltpu.VMEM((NBP,), jnp.int32),
        ],
    )(eflat, wflat)


_GROWS = NPAD // NW
_GCH = 40


def _gather_body(x_hbm, gidx_hbm, xs_hbm, idxv, r0, r1, gsem, wsem):
    base = _wid() * _GROWS
    pltpu.sync_copy(gidx_hbm.at[pl.ds(base, _GROWS)], idxv)
    bufs = (r0, r1)
    nch = _GROWS // _GCH
    g = [None] * nch
    w = [None] * nch
    g[0] = pltpu.async_copy(x_hbm.at[idxv.at[pl.ds(0, _GCH)]], bufs[0], gsem)
    for c in range(nch):
        if c + 1 < nch:
            if c >= 1:
                w[c - 1].wait()
            g[c + 1] = pltpu.async_copy(
                x_hbm.at[idxv.at[pl.ds((c + 1) * _GCH, _GCH)]],
                bufs[(c + 1) % 2], gsem)
        g[c].wait()
        w[c] = pltpu.async_copy(bufs[c % 2],
                                xs_hbm.at[pl.ds(base + c * _GCH, _GCH)], wsem)
    w[nch - 2].wait()
    w[nch - 1].wait()


def _gather(x, gidx):
    return pl.kernel(
        _gather_body,
        out_type=jax.ShapeDtypeStruct((NPAD, HS), jnp.float32),
        mesh=_SC_MESH,
        compiler_params=_SC_PARAMS,
        scratch_types=[
            pltpu.VMEM((_GROWS,), jnp.int32),
            pltpu.VMEM((_GCH, HS), jnp.float32),
            pltpu.VMEM((_GCH, HS), jnp.float32),
            pltpu.SemaphoreType.DMA,
            pltpu.SemaphoreType.DMA,
        ],
    )(x, gidx)


def _group_body(bexp_ref, xs_ref, wg_ref, wu_ref, wd_ref, ws_ref, ys_ref):
    e = bexp_ref[pl.program_id(0)]

    @pl.when(e >= 0)
    def _():
        x = xs_ref[...].astype(jnp.bfloat16)
        g = jnp.dot(x, wg_ref[0].astype(jnp.bfloat16),
                    preferred_element_type=jnp.float32)
        u = jnp.dot(x, wu_ref[0].astype(jnp.bfloat16),
                    preferred_element_type=jnp.float32)
        h = ((g * jax.nn.sigmoid(g)) * u).astype(jnp.bfloat16)
        o = jnp.dot(h, wd_ref[0].astype(jnp.bfloat16),
                    preferred_element_type=jnp.float32)
        ys_ref[...] = ws_ref[...] * o


def _grouped(bexp, xs, gate_proj_w, up_proj_w, down_proj_w, ws):
    grid_spec = pltpu.PrefetchScalarGridSpec(
        num_scalar_prefetch=1,
        grid=(NB,),
        in_specs=[
            pl.BlockSpec((BM, HS), lambda b, bexp: (b, 0)),
            pl.BlockSpec((1, HS, DFF),
                         lambda b, bexp: (jnp.maximum(bexp[b], 0), 0, 0)),
            pl.BlockSpec((1, HS, DFF),
                         lambda b, bexp: (jnp.maximum(bexp[b], 0), 0, 0)),
            pl.BlockSpec((1, DFF, HS),
                         lambda b, bexp: (jnp.maximum(bexp[b], 0), 0, 0)),
            pl.BlockSpec((BM, 1), lambda b, bexp: (b, 0)),
        ],
        out_specs=pl.BlockSpec((BM, HS), lambda b, bexp: (b, 0)),
    )
    return pl.pallas_call(
        _group_body,
        grid_spec=grid_spec,
        out_shape=jax.ShapeDtypeStruct((NPAD, HS), jnp.float32),
    )(bexp, xs, gate_proj_w, up_proj_w, down_proj_w, ws)


def _shared_body(x_ref, wg_ref, wu_ref, wd_ref, sg_ref, out_ref):
    d = pl.program_id(0)
    t = pl.program_id(1)
    nd = pl.num_programs(0)
    bt = x_ref.shape[0]

    x = x_ref[...].astype(jnp.bfloat16)
    g = jnp.dot(x, wg_ref[...].astype(jnp.bfloat16),
                preferred_element_type=jnp.float32)
    u = jnp.dot(x, wu_ref[...].astype(jnp.bfloat16),
                preferred_element_type=jnp.float32)
    h = ((g * jax.nn.sigmoid(g)) * u).astype(jnp.bfloat16)
    y = jnp.dot(h, wd_ref[...].astype(jnp.bfloat16),
                preferred_element_type=jnp.float32)
    sl = pl.ds(t * bt, bt)

    @pl.when(d == 0)
    def _():
        out_ref[sl, :] = y

    @pl.when(d > 0)
    def _():
        out_ref[sl, :] += y

    @pl.when(d == nd - 1)
    def _():
        gate = jax.nn.sigmoid(
            jnp.dot(x_ref[...], sg_ref[...],
                    preferred_element_type=jnp.float32))
        out_ref[sl, :] *= gate


def _shared(x, wg, wu, wd, sgw, bt=512, bd=1408):
    nt, nd = S // bt, DSH // bd
    return pl.pallas_call(
        _shared_body,
        grid=(nd, nt),
        in_specs=[
            pl.BlockSpec((bt, HS), lambda d, t: (t, 0)),
            pl.BlockSpec((HS, bd), lambda d, t: (0, d)),
            pl.BlockSpec((HS, bd), lambda d, t: (0, d)),
            pl.BlockSpec((bd, HS), lambda d, t: (d, 0)),
            pl.BlockSpec((HS, 1), lambda d, t: (0, 0)),
        ],
        out_specs=pl.BlockSpec((S, HS), lambda d, t: (0, 0)),
        out_shape=jax.ShapeDtypeStruct((S, HS), jnp.float32),
    )(x, wg, wu, wd, sgw)


_TPW = S // NW
_CCH = 16


def _combine_body(ys_hbm, sh_hbm, pos_hbm, out_hbm, p0v, p1v,
                  y00, y01, y10, y11, sh0, sh1, gsem, wsem):
    tb = _wid() * _TPW
    pltpu.sync_copy(pos_hbm.at[pl.ds(tb, _TPW)], p0v)
    pltpu.sync_copy(pos_hbm.at[pl.ds(S + tb, _TPW)], p1v)
    y0b = (y00, y01)
    y1b = (y10, y11)
    shb = (sh0, sh1)
    nch = _TPW // _CCH
    g = [None] * nch
    w = [None] * nch

    def issue(c):
        sl = pl.ds(c * _CCH, _CCH)
        a = pltpu.async_copy(ys_hbm.at[p0v.at[sl]], y0b[c % 2], gsem)
        b = pltpu.async_copy(ys_hbm.at[p1v.at[sl]], y1b[c % 2], gsem)
        d = pltpu.async_copy(sh_hbm.at[pl.ds(tb + c * _CCH, _CCH)],
                             shb[c % 2], gsem)
        return (a, b, d)

    g[0] = issue(0)
    for c in range(nch):
        if c + 1 < nch:
            if c >= 1:
                w[c - 1].wait()
            g[c + 1] = issue(c + 1)
        for cp in g[c]:
            cp.wait()

        def dstep(d, carry2):
            sl = pl.ds(d * L, L)
            for i in range(_CCH):
                shb[c % 2][i, sl] += y0b[c % 2][i, sl] + y1b[c % 2][i, sl]
            return carry2
        lax.fori_loop(0, HS // L, dstep, 0)
        w[c] = pltpu.async_copy(shb[c % 2],
                                out_hbm.at[pl.ds(tb + c * _CCH, _CCH)], wsem)
    w[nch - 2].wait()
    w[nch - 1].wait()


def _combine(ys, sh, pos):
    return pl.kernel(
        _combine_body,
        out_type=jax.ShapeDtypeStruct((S, HS), jnp.float32),
        mesh=_SC_MESH,
        compiler_params=_SC_PARAMS,
        scratch_types=[
            pltpu.VMEM((_TPW,), jnp.int32),
            pltpu.VMEM((_TPW,), jnp.int32),
            pltpu.VMEM((_CCH, HS), jnp.float32),
            pltpu.VMEM((_CCH, HS), jnp.float32),
            pltpu.VMEM((_CCH, HS), jnp.float32),
            pltpu.VMEM((_CCH, HS), jnp.float32),
            pltpu.VMEM((_CCH, HS), jnp.float32),
            pltpu.VMEM((_CCH, HS), jnp.float32),
            pltpu.SemaphoreType.DMA,
            pltpu.SemaphoreType.DMA,
        ],
    )(ys, sh, pos)


def kernel(hidden_states, gate_w, gate_proj_w, up_proj_w, down_proj_w,
           shared_gate_proj_w, shared_up_proj_w, shared_down_proj_w,
           shared_expert_gate_w):
    x = hidden_states.reshape(S, HS)
    logits, w2, e2 = _router(x, gate_w)
    eflat = e2.T.reshape(NPAIR)
    wflat = w2.T.reshape(NPAIR)
    gidx, ws, pos, bexp = _meta(eflat, wflat)
    xs = _gather(x, gidx)
    sh = _shared(x, shared_gate_proj_w, shared_up_proj_w, shared_down_proj_w,
                 shared_expert_gate_w)
    ys = _grouped(bexp, xs, gate_proj_w, up_proj_w, down_proj_w,
                  ws.reshape(NPAD, 1))
    out = _combine(ys, sh, pos)
    return (out.reshape(1, S, HS), logits.reshape(1, S, E))
